# Initial kernel scaffold; baseline (speedup 1.0000x reference)
import jax
import jax.numpy as jnp
from jax.experimental import pallas as pl


def kernel(x, token_table, pos_table):
    raise NotImplementedError("write your pallas kernel here")


# speedup vs baseline: 1.2537x; 1.2537x over previous
import functools

import jax
import jax.numpy as jnp
import numpy as np
from jax import lax
from jax.experimental import pallas as pl
from jax.experimental.pallas import tpu as pltpu
from jax.experimental.pallas import tpu_sc as plsc

VOCAB = 1000000
MAXLEN = 200
EMBED = 32
BATCH = 4096
SEQ = 200

NC = 2
NS = 16
NW = NC * NS

TOTAL = BATCH * SEQ
ROWS_PER_W = TOTAL // NW
CHUNK = 1600
NCHUNK = ROWS_PER_W // CHUNK
SUB = 64
NSUB = CHUNK // SUB
SUBP = 40

_POSITIONS = np.array([0, 0] + list(range(2, 200)), dtype=np.int32)


def _body(x_hbm, positions_hbm, token_hbm, pos_table_hbm, out_hbm,
          idx_v, rows_v, pos_idx_v, pos_v, sem):
    wid = lax.axis_index("s") * NC + lax.axis_index("c")
    base0 = wid * ROWS_PER_W

    pltpu.sync_copy(positions_hbm, pos_idx_v)
    pos_copies = [
        pltpu.async_copy(pos_table_hbm.at[pos_idx_v.at[pl.ds(j * SUBP, SUBP)]],
                         pos_v.at[pl.ds(j * SUBP, SUBP)], sem)
        for j in range(MAXLEN // SUBP)
    ]
    for cp in pos_copies:
        cp.wait()

    def chunk_body(c, carry):
        base = base0 + c * CHUNK
        pltpu.sync_copy(x_hbm.at[pl.ds(base, CHUNK)], idx_v)
        copies = [
            pltpu.async_copy(token_hbm.at[idx_v.at[pl.ds(j * SUB, SUB)]],
                             rows_v.at[pl.ds(j * SUB, SUB)], sem)
            for j in range(NSUB)
        ]
        for cp in copies:
            cp.wait()

        def seq_body(s, carry2):
            row0 = s * MAXLEN
            for r in range(MAXLEN):
                for h in (0, 1):
                    pv = pos_v[r, pl.ds(h * 16, 16)]
                    plsc.addupdate(rows_v.at[row0 + r, pl.ds(h * 16, 16)], pv)
            return carry2

        lax.fori_loop(0, CHUNK // MAXLEN, seq_body, 0)

        pltpu.sync_copy(rows_v, out_hbm.at[pl.ds(base, CHUNK)])
        return carry

    lax.fori_loop(0, NCHUNK, chunk_body, 0)


@jax.jit
def kernel(x, token_table, pos_table):
    x_flat = x.reshape(TOTAL).astype(jnp.int32)
    positions = jnp.asarray(_POSITIONS)

    run = pl.kernel(
        _body,
        out_type=jax.ShapeDtypeStruct((TOTAL, EMBED), jnp.float32),
        mesh=plsc.VectorSubcoreMesh(core_axis_name="c", subcore_axis_name="s"),
        compiler_params=pltpu.CompilerParams(use_tc_tiling_on_sc=False),
        scratch_types=[
            pltpu.VMEM((CHUNK,), jnp.int32),
            pltpu.VMEM((CHUNK, EMBED), jnp.float32),
            pltpu.VMEM((MAXLEN,), jnp.int32),
            pltpu.VMEM((MAXLEN, EMBED), jnp.float32),
            pltpu.SemaphoreType.DMA,
        ],
    )
    out = run(x_flat, positions, token_table, pos_table)
    return out.reshape(BATCH, SEQ, EMBED)
